# baseline (speedup 1.0000x reference)
import jax
import jax.numpy as jnp
from jax import lax
from jax.experimental import pallas as pl
from jax.experimental.pallas import tpu as pltpu
from jax.experimental.pallas import tpu_sc as plsc

_NC = 2
_NS = 16
_NW = _NC * _NS


def _sc_copy_body(table_hbm, out_hbm, sem):
    rows = out_hbm.shape[0]
    rows_per_worker = rows // _NW
    wid = lax.axis_index("s") * _NC + lax.axis_index("c")
    base = wid * rows_per_worker
    pltpu.make_async_copy(
        table_hbm.at[pl.ds(base, rows_per_worker), :],
        out_hbm.at[pl.ds(base, rows_per_worker), :],
        sem).start()
    pltpu.make_async_copy(
        table_hbm.at[pl.ds(base, rows_per_worker), :],
        out_hbm.at[pl.ds(base, rows_per_worker), :],
        sem).wait()


def kernel(x, table):
    T = x.shape[1]
    _, d = table.shape
    mesh = plsc.VectorSubcoreMesh(core_axis_name="c", subcore_axis_name="s")
    f = pl.kernel(
        _sc_copy_body,
        mesh=mesh,
        out_type=jax.ShapeDtypeStruct((T, d), table.dtype),
        scratch_types=[pltpu.SemaphoreType.DMA],
    )
    return f(table)


# speedup vs baseline: 23.8490x; 23.8490x over previous
import jax
import jax.numpy as jnp
from jax import lax
from jax.experimental import pallas as pl
from jax.experimental.pallas import tpu as pltpu
from jax.experimental.pallas import tpu_sc as plsc

_NC = 2
_NS = 16
_NW = _NC * _NS
_CHUNK_ROWS = 16
_NBUF = 3


def _sc_copy_body(table_hbm, out_hbm, buf0, buf1, buf2, isem, osem):
    rows = out_hbm.shape[0]
    rows_per_worker = rows // _NW
    n_chunks = rows_per_worker // _CHUNK_ROWS
    wid = lax.axis_index("s") * _NC + lax.axis_index("c")
    base = wid * rows_per_worker
    bufs = (buf0, buf1, buf2)

    def in_copy(c):
        return pltpu.make_async_copy(
            table_hbm.at[pl.ds(base + c * _CHUNK_ROWS, _CHUNK_ROWS), :],
            bufs[c % _NBUF], isem.at[c % _NBUF])

    def out_copy(c):
        return pltpu.make_async_copy(
            bufs[c % _NBUF],
            out_hbm.at[pl.ds(base + c * _CHUNK_ROWS, _CHUNK_ROWS), :],
            osem.at[c % _NBUF])

    in_copy(0).start()
    in_copy(1).start()
    for c in range(n_chunks):
        in_copy(c).wait()
        out_copy(c).start()
        if c + 2 < n_chunks:
            if c >= 1:
                out_copy(c - 1).wait()
            in_copy(c + 2).start()
    for c in range(max(0, n_chunks - 3), n_chunks):
        out_copy(c).wait()


def kernel(x, table):
    T = x.shape[1]
    _, d = table.shape
    mesh = plsc.VectorSubcoreMesh(core_axis_name="c", subcore_axis_name="s")
    f = pl.kernel(
        _sc_copy_body,
        mesh=mesh,
        out_type=jax.ShapeDtypeStruct((T, d), table.dtype),
        scratch_types=[
            pltpu.VMEM((_CHUNK_ROWS, d), table.dtype),
            pltpu.VMEM((_CHUNK_ROWS, d), table.dtype),
            pltpu.VMEM((_CHUNK_ROWS, d), table.dtype),
            pltpu.SemaphoreType.DMA((_NBUF,)),
            pltpu.SemaphoreType.DMA((_NBUF,)),
        ],
    )
    return f(table)


# speedup vs baseline: 49.0987x; 2.0587x over previous
import jax
import jax.numpy as jnp
from jax.experimental import pallas as pl

_ROWS_PER_BLOCK = 1024


def _copy_block(t_ref, o_ref):
    o_ref[...] = t_ref[...]


def kernel(x, table):
    T = x.shape[1]
    _, d = table.shape
    grid = (T // _ROWS_PER_BLOCK,)
    return pl.pallas_call(
        _copy_block,
        grid=grid,
        in_specs=[pl.BlockSpec((_ROWS_PER_BLOCK, d), lambda i: (i, 0))],
        out_specs=pl.BlockSpec((_ROWS_PER_BLOCK, d), lambda i: (i, 0)),
        out_shape=jax.ShapeDtypeStruct((T, d), table.dtype),
    )(table)
